# baseline (speedup 1.0000x reference)
import jax
import jax.numpy as jnp
from jax.experimental import pallas as pl
from jax.experimental.pallas import tpu as pltpu

_N = 1024
_LATENT = 64
_HIDDEN = 256
_OUT = 2000
_NUM_OMICS = 3
_EPS = 1e-3
_ROWS = 8
_W = 512
_K = -(-_OUT // _W)


def _moum_kernel(at_ref, x_ref, w1_ref, b1_ref, w2_ref, b2_ref,
                 wg1_ref, bg1_ref, g1_ref, be1_ref,
                 wg2_ref, bg2_ref, g2_ref, be2_ref, out_ref, h_ref):
    i = pl.program_id(0)
    inv = 1.0 / jnp.sqrt(1.0 + _EPS)

    @pl.when(i == 0)
    def _gcn_and_hidden():
        at = at_ref[...].astype(jnp.float32)
        deg = jnp.sum(at, axis=1, keepdims=True) + 1.0
        norm = jax.lax.rsqrt(jnp.maximum(deg, 1.0))
        x = x_ref[...]
        y = x * norm
        z = jnp.dot(at, y, preferred_element_type=jnp.float32) + y
        agg = z * norm
        x1 = jnp.maximum(
            jnp.dot(agg, w1_ref[...], preferred_element_type=jnp.float32) + b1_ref[...],
            0.0)
        y1 = x1 * norm
        z2 = jnp.dot(at[0:_ROWS, :], y1, preferred_element_type=jnp.float32) + y1[0:_ROWS, :]
        agg2 = z2 * norm[0:_ROWS, :]
        x2 = jnp.maximum(
            jnp.dot(agg2, w2_ref[...], preferred_element_type=jnp.float32) + b2_ref[...],
            0.0)
        for o in range(_NUM_OMICS):
            h = jnp.dot(x2[o:o + 1, :], wg1_ref[o],
                        preferred_element_type=jnp.float32) + bg1_ref[o:o + 1, :]
            h = g1_ref[o:o + 1, :] * h * inv + be1_ref[o:o + 1, :]
            h_ref[o:o + 1, :] = jnp.maximum(h, 0.0)

    for o in range(_NUM_OMICS):
        w = wg2_ref[o * _HIDDEN:(o + 1) * _HIDDEN, :]
        oo = jnp.dot(h_ref[o:o + 1, :], w,
                     preferred_element_type=jnp.float32) + bg2_ref[o:o + 1, :]
        out_ref[o:o + 1, :] = g2_ref[o:o + 1, :] * oo * inv + be2_ref[o:o + 1, :]


def kernel(latent_vectors, adjacency_matrix, W_gnn1, b_gnn1, W_gnn2, b_gnn2,
           Wg1, bg1, gamma1, beta1, Wg2, bg2, gamma2, beta2):
    at = adjacency_matrix.T.astype(jnp.int8)
    wg2r = Wg2.reshape(_NUM_OMICS * _HIDDEN, _OUT)
    fix = lambda *b: pl.BlockSpec(b, lambda i: tuple(0 for _ in b))
    return pl.pallas_call(
        _moum_kernel,
        grid=(_K,),
        in_specs=[
            fix(_N, _N), fix(_N, _LATENT),
            fix(_LATENT, _LATENT), fix(1, _LATENT), fix(_LATENT, _LATENT), fix(1, _LATENT),
            fix(_NUM_OMICS, _LATENT, _HIDDEN), fix(_NUM_OMICS, _HIDDEN),
            fix(_NUM_OMICS, _HIDDEN), fix(_NUM_OMICS, _HIDDEN),
            pl.BlockSpec((_NUM_OMICS * _HIDDEN, _W), lambda i: (0, i)),
            pl.BlockSpec((_NUM_OMICS, _W), lambda i: (0, i)),
            pl.BlockSpec((_NUM_OMICS, _W), lambda i: (0, i)),
            pl.BlockSpec((_NUM_OMICS, _W), lambda i: (0, i)),
        ],
        out_specs=pl.BlockSpec((_NUM_OMICS, _W), lambda i: (0, i)),
        out_shape=jax.ShapeDtypeStruct((_NUM_OMICS, _OUT), jnp.float32),
        scratch_shapes=[pltpu.VMEM((_ROWS, _HIDDEN), jnp.float32)],
    )(at, latent_vectors,
      W_gnn1, b_gnn1.reshape(1, _LATENT), W_gnn2, b_gnn2.reshape(1, _LATENT),
      Wg1, bg1, gamma1, beta1, wg2r, bg2, gamma2, beta2)


# speedup vs baseline: 1.1150x; 1.1150x over previous
import jax
import jax.numpy as jnp
from jax.experimental import pallas as pl
from jax.experimental.pallas import tpu as pltpu

_N = 1024
_LATENT = 64
_HIDDEN = 256
_OUT = 2000
_NUM_OMICS = 3
_EPS = 1e-3
_ROWS = 8


def _moum_kernel(at_ref, x_ref, w1_ref, b1_ref, w2_ref, b2_ref,
                 wg1_ref, bg1_ref, g1_ref, be1_ref,
                 wg2_ref, bg2_ref, g2_ref, be2_ref, out_ref, x2_ref):
    i = pl.program_id(0)
    inv = 1.0 / jnp.sqrt(1.0 + _EPS)

    @pl.when(i == 0)
    def _gcn():
        at = at_ref[...].astype(jnp.float32)
        deg = jnp.sum(at, axis=1, keepdims=True) + 1.0
        norm = jax.lax.rsqrt(jnp.maximum(deg, 1.0))
        x = x_ref[...]
        y = x * norm
        z = jnp.dot(at, y, preferred_element_type=jnp.float32) + y
        agg = z * norm
        x1 = jnp.maximum(
            jnp.dot(agg, w1_ref[...], preferred_element_type=jnp.float32) + b1_ref[...],
            0.0)
        y1 = x1 * norm
        z2 = jnp.dot(at[0:_ROWS, :], y1, preferred_element_type=jnp.float32) + y1[0:_ROWS, :]
        agg2 = z2 * norm[0:_ROWS, :]
        x2_ref[...] = jnp.maximum(
            jnp.dot(agg2, w2_ref[...], preferred_element_type=jnp.float32) + b2_ref[...],
            0.0)

    row = x2_ref[pl.ds(i, 1), :]
    hh = jnp.dot(row, wg1_ref[0], preferred_element_type=jnp.float32) + bg1_ref[0]
    hh = g1_ref[0] * hh * inv + be1_ref[0]
    hh = jnp.maximum(hh, 0.0)
    oo = jnp.dot(hh, wg2_ref[0], preferred_element_type=jnp.float32) + bg2_ref[0]
    out_ref[0] = g2_ref[0] * oo * inv + be2_ref[0]


def kernel(latent_vectors, adjacency_matrix, W_gnn1, b_gnn1, W_gnn2, b_gnn2,
           Wg1, bg1, gamma1, beta1, Wg2, bg2, gamma2, beta2):
    at = adjacency_matrix.T.astype(jnp.int8)
    fix = lambda *b: pl.BlockSpec(b, lambda i: tuple(0 for _ in b))
    step = lambda *b: pl.BlockSpec((1,) + b, lambda i: (i,) + tuple(0 for _ in b))
    out = pl.pallas_call(
        _moum_kernel,
        grid=(_NUM_OMICS,),
        in_specs=[
            fix(_N, _N), fix(_N, _LATENT),
            fix(_LATENT, _LATENT), fix(1, _LATENT), fix(_LATENT, _LATENT), fix(1, _LATENT),
            step(_LATENT, _HIDDEN), step(1, _HIDDEN), step(1, _HIDDEN), step(1, _HIDDEN),
            step(_HIDDEN, _OUT), step(1, _OUT), step(1, _OUT), step(1, _OUT),
        ],
        out_specs=pl.BlockSpec((1, 1, _OUT), lambda i: (i, 0, 0)),
        out_shape=jax.ShapeDtypeStruct((_NUM_OMICS, 1, _OUT), jnp.float32),
        scratch_shapes=[pltpu.VMEM((_ROWS, _LATENT), jnp.float32)],
    )(at, latent_vectors,
      W_gnn1, b_gnn1.reshape(1, _LATENT), W_gnn2, b_gnn2.reshape(1, _LATENT),
      Wg1, bg1.reshape(_NUM_OMICS, 1, _HIDDEN),
      gamma1.reshape(_NUM_OMICS, 1, _HIDDEN), beta1.reshape(_NUM_OMICS, 1, _HIDDEN),
      Wg2, bg2.reshape(_NUM_OMICS, 1, _OUT),
      gamma2.reshape(_NUM_OMICS, 1, _OUT), beta2.reshape(_NUM_OMICS, 1, _OUT))
    return out.reshape(_NUM_OMICS, _OUT)
